# Initial kernel scaffold; baseline (speedup 1.0000x reference)
import functools

import jax
import jax.numpy as jnp
from jax import lax
from jax.experimental import pallas as pl
from jax.experimental.pallas import tpu as pltpu
from jax.experimental.pallas import tpu_sc as plsc

N = 10000
K = 32
D = 128
NW = 32
B = 4
NP = 10240
G = NP // B
GPW = G // NW
LANES = 16


def _sc_weighted_gather_sum(inp, idx_flat, c_pad):
    mesh = plsc.VectorSubcoreMesh(core_axis_name="c", subcore_axis_name="s")

    @functools.partial(
        pl.kernel,
        mesh=mesh,
        out_type=jax.ShapeDtypeStruct((NP, D), jnp.float32),
        scratch_types=[
            pltpu.VMEM((B * K,), jnp.int32),
            pltpu.VMEM((B * K, D), jnp.float32),
            pltpu.VMEM((B, K), jnp.float32),
            pltpu.VMEM((B, D), jnp.float32),
            pltpu.SemaphoreType.DMA,
        ],
    )
    def sc_kernel(inp_hbm, idx_hbm, c_hbm, out_hbm, idx_v, rows_v, c_v, out_v, sem):
        wid = lax.axis_index("s") * 2 + lax.axis_index("c")
        g0 = wid * GPW

        def body(i, _):
            g = g0 + i
            pltpu.sync_copy(idx_hbm.at[pl.ds(g * (B * K), B * K)], idx_v)
            pltpu.sync_copy(c_hbm.at[pl.ds(g * B, B)], c_v)
            pltpu.async_copy(inp_hbm.at[idx_v], rows_v, sem).wait()
            for r in range(B):
                cvals = [c_v[r, kk] for kk in range(K)]
                for ch in range(D // LANES):
                    acc = cvals[0] * rows_v[r * K, pl.ds(ch * LANES, LANES)]
                    for kk in range(1, K):
                        acc = acc + cvals[kk] * rows_v[r * K + kk, pl.ds(ch * LANES, LANES)]
                    out_v[r, pl.ds(ch * LANES, LANES)] = acc
            pltpu.sync_copy(out_v, out_hbm.at[pl.ds(g * B, B)])
            return ()

        lax.fori_loop(0, GPW, body, ())

    return sc_kernel(inp, idx_flat, c_pad)


def _tc_body(s_ref, c_ref, wt_ref, b_ref, o_ref):
    s = s_ref[...]
    cs = jnp.sum(c_ref[...], axis=1, keepdims=True)
    o = jnp.dot(s, wt_ref[...], preferred_element_type=jnp.float32)
    o_ref[...] = jax.nn.sigmoid(o + cs * b_ref[...])


def _tc_linear_sigmoid(s, c_pad, wt, b2d):
    bn = 640
    grid = NP // bn
    return pl.pallas_call(
        _tc_body,
        grid=(grid,),
        in_specs=[
            pl.BlockSpec((bn, D), lambda i: (i, 0)),
            pl.BlockSpec((bn, K), lambda i: (i, 0)),
            pl.BlockSpec((D, D), lambda i: (0, 0)),
            pl.BlockSpec((1, D), lambda i: (0, 0)),
        ],
        out_specs=pl.BlockSpec((bn, D), lambda i: (i, 0)),
        out_shape=jax.ShapeDtypeStruct((NP, D), jnp.float32),
    )(s, c_pad, wt, b2d)


def kernel(Input, I, C, W, b):
    idx = I.astype(jnp.int32).reshape(-1)
    idx_flat = jnp.pad(idx, (0, (NP - N) * K))
    c_pad = jnp.pad(C, ((0, NP - N), (0, 0)))
    s = _sc_weighted_gather_sum(Input, idx_flat, c_pad)
    out = _tc_linear_sigmoid(s, c_pad, W.T, b.reshape(1, D))
    return out[:N]


# speedup vs baseline: 1.4719x; 1.4719x over previous
import functools

import jax
import jax.numpy as jnp
from jax import lax
from jax.experimental import pallas as pl
from jax.experimental.pallas import tpu as pltpu
from jax.experimental.pallas import tpu_sc as plsc

N = 10000
K = 32
D = 128
NW = 32
B = 4
NP = 10240
G = NP // B
GPW = G // NW
LANES = 16


def _sc_weighted_gather_sum(inp, idx_flat, c_pad):
    mesh = plsc.VectorSubcoreMesh(core_axis_name="c", subcore_axis_name="s")

    @functools.partial(
        pl.kernel,
        mesh=mesh,
        out_type=jax.ShapeDtypeStruct((NP, D), jnp.float32),
        scratch_types=[
            pltpu.VMEM((B * K,), jnp.int32),
            pltpu.VMEM((B * K, D), jnp.float32),
            pltpu.VMEM((B, K), jnp.float32),
            pltpu.VMEM((B, D), jnp.float32),
            pltpu.SemaphoreType.DMA,
        ],
    )
    def sc_kernel(inp_hbm, idx_hbm, c_hbm, out_hbm, idx_v, rows_v, c_v, out_v, sem):
        wid = lax.axis_index("s") * 2 + lax.axis_index("c")
        g0 = wid * GPW

        def body(i, _):
            g = g0 + i
            pltpu.sync_copy(idx_hbm.at[pl.ds(g * (B * K), B * K)], idx_v)
            pltpu.sync_copy(c_hbm.at[pl.ds(g * B, B)], c_v)
            pltpu.async_copy(inp_hbm.at[idx_v], rows_v, sem).wait()
            for r in range(B):
                cvecs = [c_v[r, pl.ds(h * LANES, LANES)] for h in range(K // LANES)]
                cvals = [cvecs[kk // LANES][kk % LANES] for kk in range(K)]
                for ch in range(D // LANES):
                    acc = cvals[0] * rows_v[r * K, pl.ds(ch * LANES, LANES)]
                    for kk in range(1, K):
                        acc = acc + cvals[kk] * rows_v[r * K + kk, pl.ds(ch * LANES, LANES)]
                    out_v[r, pl.ds(ch * LANES, LANES)] = acc
            pltpu.sync_copy(out_v, out_hbm.at[pl.ds(g * B, B)])
            return ()

        lax.fori_loop(0, GPW, body, ())

    return sc_kernel(inp, idx_flat, c_pad)


def _tc_body(s_ref, c_ref, wt_ref, b_ref, o_ref):
    s = s_ref[...]
    cs = jnp.sum(c_ref[...], axis=1, keepdims=True)
    o = jnp.dot(s, wt_ref[...], preferred_element_type=jnp.float32)
    o_ref[...] = jax.nn.sigmoid(o + cs * b_ref[...])


def _tc_linear_sigmoid(s, c_pad, wt, b2d):
    bn = 640
    grid = NP // bn
    return pl.pallas_call(
        _tc_body,
        grid=(grid,),
        in_specs=[
            pl.BlockSpec((bn, D), lambda i: (i, 0)),
            pl.BlockSpec((bn, K), lambda i: (i, 0)),
            pl.BlockSpec((D, D), lambda i: (0, 0)),
            pl.BlockSpec((1, D), lambda i: (0, 0)),
        ],
        out_specs=pl.BlockSpec((bn, D), lambda i: (i, 0)),
        out_shape=jax.ShapeDtypeStruct((NP, D), jnp.float32),
    )(s, c_pad, wt, b2d)


def kernel(Input, I, C, W, b):
    idx = I.astype(jnp.int32).reshape(-1)
    idx_flat = jnp.pad(idx, (0, (NP - N) * K))
    c_pad = jnp.pad(C, ((0, NP - N), (0, 0)))
    s = _sc_weighted_gather_sum(Input, idx_flat, c_pad)
    out = _tc_linear_sigmoid(s, c_pad, W.T, b.reshape(1, D))
    return out[:N]


# speedup vs baseline: 2.1574x; 1.4657x over previous
import functools

import jax
import jax.numpy as jnp
from jax import lax
from jax.experimental import pallas as pl
from jax.experimental.pallas import tpu as pltpu
from jax.experimental.pallas import tpu_sc as plsc

N = 10000
K = 32
D = 128
NW = 32
B = 4
NP = 10240
G = NP // B
GPW = G // NW
LANES = 16


def _sc_weighted_gather_sum(inp, idx_flat, c_pad):
    mesh = plsc.VectorSubcoreMesh(core_axis_name="c", subcore_axis_name="s")

    nbuf = 2
    rpw = GPW * B

    @functools.partial(
        pl.kernel,
        mesh=mesh,
        out_type=jax.ShapeDtypeStruct((NP, D), jnp.float32),
        scratch_types=[
            pltpu.VMEM((GPW, B * K), jnp.int32),
            pltpu.VMEM((rpw, K), jnp.float32),
            [pltpu.VMEM((B * K, D), jnp.float32) for _ in range(nbuf)],
            [pltpu.VMEM((B, D), jnp.float32) for _ in range(nbuf)],
            [pltpu.SemaphoreType.DMA for _ in range(nbuf)],
            [pltpu.SemaphoreType.DMA for _ in range(nbuf)],
        ],
    )
    def sc_kernel(inp_hbm, idx_hbm, c_hbm, out_hbm, idx_v, c_v, rows, outs, sems, osems):
        wid = lax.axis_index("s") * 2 + lax.axis_index("c")
        g0 = wid * GPW

        pltpu.sync_copy(idx_hbm.at[pl.ds(g0, GPW)], idx_v)
        pltpu.sync_copy(c_hbm.at[pl.ds(g0 * B, rpw)], c_v)

        def fire(i, b):
            pltpu.async_copy(inp_hbm.at[idx_v.at[i]], rows[b], sems[b])

        def out_copy(i, b):
            return pltpu.make_async_copy(outs[b], out_hbm.at[pl.ds((g0 + i) * B, B)],
                                         osems[b])

        def compute(i, b):
            rows_v = rows[b]
            for r in range(B):
                cvecs = [c_v[i * B + r, pl.ds(h * LANES, LANES)]
                         for h in range(K // LANES)]
                cvals = [cvecs[kk // LANES][kk % LANES] for kk in range(K)]
                for ch in range(D // LANES):
                    acc = cvals[0] * rows_v[r * K, pl.ds(ch * LANES, LANES)]
                    for kk in range(1, K):
                        acc = acc + cvals[kk] * rows_v[r * K + kk, pl.ds(ch * LANES, LANES)]
                    outs[b][r, pl.ds(ch * LANES, LANES)] = acc

        for b in range(nbuf):
            fire(b, b)

        def body(j, _):
            for b in range(nbuf):
                i = nbuf * j + b
                pltpu.make_async_copy(inp_hbm.at[idx_v.at[i]], rows[b], sems[b]).wait()
                pl.when(j > 0)(lambda: out_copy(i, b).wait())
                compute(i, b)
                pl.when(i + nbuf < GPW)(lambda: fire(i + nbuf, b))
                out_copy(i, b).start()
            return ()

        lax.fori_loop(0, GPW // nbuf, body, ())
        for b in range(nbuf):
            out_copy(GPW - nbuf + b, b).wait()

    return sc_kernel(inp, idx_flat, c_pad)


def _tc_body(s_ref, c_ref, wt_ref, b_ref, o_ref):
    s = s_ref[...]
    cs = jnp.sum(c_ref[...], axis=1, keepdims=True)
    o = jnp.dot(s, wt_ref[...], preferred_element_type=jnp.float32)
    o_ref[...] = jax.nn.sigmoid(o + cs * b_ref[...])


def _tc_linear_sigmoid(s, c_pad, wt, b2d):
    bn = 640
    grid = NP // bn
    return pl.pallas_call(
        _tc_body,
        grid=(grid,),
        in_specs=[
            pl.BlockSpec((bn, D), lambda i: (i, 0)),
            pl.BlockSpec((bn, K), lambda i: (i, 0)),
            pl.BlockSpec((D, D), lambda i: (0, 0)),
            pl.BlockSpec((1, D), lambda i: (0, 0)),
        ],
        out_specs=pl.BlockSpec((bn, D), lambda i: (i, 0)),
        out_shape=jax.ShapeDtypeStruct((NP, D), jnp.float32),
    )(s, c_pad, wt, b2d)


def kernel(Input, I, C, W, b):
    idx = I.astype(jnp.int32).reshape(-1)
    idx_flat = jnp.pad(idx, (0, (NP - N) * K)).reshape(G, B * K)
    c_pad = jnp.pad(C, ((0, NP - N), (0, 0)))
    s = _sc_weighted_gather_sum(Input, idx_flat, c_pad)
    out = _tc_linear_sigmoid(s, c_pad, W.T, b.reshape(1, D))
    return out[:N]


# speedup vs baseline: 2.1661x; 1.0041x over previous
import functools

import jax
import jax.numpy as jnp
from jax import lax
from jax.experimental import pallas as pl
from jax.experimental.pallas import tpu as pltpu
from jax.experimental.pallas import tpu_sc as plsc

N = 10000
K = 32
D = 128
NW = 32
B = 2
NP = 10240
G = NP // B
GPW = G // NW
LANES = 16


def _sc_weighted_gather_sum(inp, idx_flat, c_pad):
    mesh = plsc.VectorSubcoreMesh(core_axis_name="c", subcore_axis_name="s")

    nbuf = 4
    rpw = GPW * B

    @functools.partial(
        pl.kernel,
        mesh=mesh,
        out_type=jax.ShapeDtypeStruct((NP, D), jnp.float32),
        scratch_types=[
            pltpu.VMEM((GPW, B * K), jnp.int32),
            pltpu.VMEM((rpw, K), jnp.float32),
            [pltpu.VMEM((B * K, D), jnp.float32) for _ in range(nbuf)],
            [pltpu.VMEM((B, D), jnp.float32) for _ in range(nbuf)],
            [pltpu.SemaphoreType.DMA for _ in range(nbuf)],
            [pltpu.SemaphoreType.DMA for _ in range(nbuf)],
        ],
    )
    def sc_kernel(inp_hbm, idx_hbm, c_hbm, out_hbm, idx_v, c_v, rows, outs, sems, osems):
        wid = lax.axis_index("s") * 2 + lax.axis_index("c")
        g0 = wid * GPW

        pltpu.sync_copy(idx_hbm.at[pl.ds(g0, GPW)], idx_v)
        pltpu.sync_copy(c_hbm.at[pl.ds(g0 * B, rpw)], c_v)

        def fire(i, b):
            pltpu.async_copy(inp_hbm.at[idx_v.at[i]], rows[b], sems[b])

        def out_copy(i, b):
            return pltpu.make_async_copy(outs[b], out_hbm.at[pl.ds((g0 + i) * B, B)],
                                         osems[b])

        def compute(i, b):
            rows_v = rows[b]
            for r in range(B):
                cvecs = [c_v[i * B + r, pl.ds(h * LANES, LANES)]
                         for h in range(K // LANES)]
                cvals = [cvecs[kk // LANES][kk % LANES] for kk in range(K)]
                for ch in range(D // LANES):
                    acc = cvals[0] * rows_v[r * K, pl.ds(ch * LANES, LANES)]
                    for kk in range(1, K):
                        acc = acc + cvals[kk] * rows_v[r * K + kk, pl.ds(ch * LANES, LANES)]
                    outs[b][r, pl.ds(ch * LANES, LANES)] = acc

        for b in range(nbuf):
            fire(b, b)

        def body(j, _):
            for b in range(nbuf):
                i = nbuf * j + b
                pltpu.make_async_copy(inp_hbm.at[idx_v.at[i]], rows[b], sems[b]).wait()
                pl.when(j > 0)(lambda: out_copy(i, b).wait())
                compute(i, b)
                pl.when(i + nbuf < GPW)(lambda: fire(i + nbuf, b))
                out_copy(i, b).start()
            return ()

        lax.fori_loop(0, GPW // nbuf, body, ())
        for b in range(nbuf):
            out_copy(GPW - nbuf + b, b).wait()

    return sc_kernel(inp, idx_flat, c_pad)


def _tc_body(s_ref, c_ref, wt_ref, b_ref, o_ref):
    s = s_ref[...]
    cs = jnp.sum(c_ref[...], axis=1, keepdims=True)
    o = jnp.dot(s, wt_ref[...], preferred_element_type=jnp.float32)
    o_ref[...] = jax.nn.sigmoid(o + cs * b_ref[...])


def _tc_linear_sigmoid(s, c_pad, wt, b2d):
    bn = 640
    grid = NP // bn
    return pl.pallas_call(
        _tc_body,
        grid=(grid,),
        in_specs=[
            pl.BlockSpec((bn, D), lambda i: (i, 0)),
            pl.BlockSpec((bn, K), lambda i: (i, 0)),
            pl.BlockSpec((D, D), lambda i: (0, 0)),
            pl.BlockSpec((1, D), lambda i: (0, 0)),
        ],
        out_specs=pl.BlockSpec((bn, D), lambda i: (i, 0)),
        out_shape=jax.ShapeDtypeStruct((NP, D), jnp.float32),
    )(s, c_pad, wt, b2d)


def kernel(Input, I, C, W, b):
    idx = I.astype(jnp.int32).reshape(-1)
    idx_flat = jnp.pad(idx, (0, (NP - N) * K)).reshape(G, B * K)
    c_pad = jnp.pad(C, ((0, NP - N), (0, 0)))
    s = _sc_weighted_gather_sum(Input, idx_flat, c_pad)
    out = _tc_linear_sigmoid(s, c_pad, W.T, b.reshape(1, D))
    return out[:N]
